# Initial kernel scaffold; baseline (speedup 1.0000x reference)
import jax
import jax.numpy as jnp
from jax.experimental import pallas as pl


def kernel(filtration_values, finite_indices, essential_indices):
    raise NotImplementedError("write your pallas kernel here")


# speedup vs baseline: 1.6905x; 1.6905x over previous
import functools
import jax
import jax.numpy as jnp
from jax import lax
from jax.experimental import pallas as pl
from jax.experimental.pallas import tpu as pltpu
from jax.experimental.pallas import tpu_sc as plsc

N = 100000
P = 50000
NW = 16
CPW = 3200
VPW = 2 * CPW
NCH = VPW // 128
RED = 6272
NPAD = NW * RED
IPAD = NW * VPW
OWNER = P // CPW
LSLOT = P - OWNER * CPW
F32 = jnp.float32
I32 = jnp.int32


def _perm(v, idx):
    return v.at[idx].get(mode="promise_in_bounds")


def _bcast(v, i):
    return _perm(v, jnp.zeros((16,), I32) + i)


def _bfly(v, op):
    iot = lax.iota(I32, 16)
    for s in (1, 2, 4, 8):
        v = op(v, _perm(v, iot ^ s))
    return v


def _body(tab, fidx, eidx, ofin, oess,
          idx_v, vals_v, sidx_v, fil_v, mbuf, ebuf, fillb,
          ev_i, ev_o, pub_v, lpart_v, pubc_v, lcnt_v,
          sh_f, sh_c, gsem, ssem, esem):
    wid = lax.axis_index("s")
    iot = lax.iota(I32, 16)
    z = iot * 0

    pltpu.sync_copy(fidx.at[pl.ds(wid * VPW, VPW)], idx_v)

    def fire_g(j, c):
        pltpu.async_copy(tab.at[idx_v.at[pl.ds(j * 128, 128)]],
                         vals_v.at[pl.ds(j * 128, 128)], gsem)
        return c
    lax.fori_loop(0, NCH, fire_g, 0)

    @pl.when(wid == 0)
    def _():
        pltpu.sync_copy(eidx, ev_i)
        pltpu.async_copy(tab.at[ev_i], ev_o, esem)

    pltpu.sync_copy(tab.at[pl.ds(wid * RED, RED)], fil_v)

    def red(j, c):
        mn, mx = c
        v = fil_v[pl.ds(j * 16, 16)]
        ok = (wid * RED + j * 16 + iot) < N
        mn = jnp.minimum(mn, jnp.where(ok, v, jnp.inf))
        mx = jnp.maximum(mx, jnp.where(ok, v, -jnp.inf))
        return mn, mx
    mn, mx = lax.fori_loop(0, RED // 16, red,
                           (jnp.full((16,), jnp.inf, F32),
                            jnp.full((16,), -jnp.inf, F32)))
    min_v = _bfly(mn, jnp.minimum)
    max_v = _bfly(mx, jnp.maximum)

    pltpu.make_async_copy(tab.at[pl.ds(0, VPW)], vals_v, gsem).wait()

    v0 = vals_v[pl.ds(0, 16)]
    row = jnp.where(iot == 0, min_v,
          jnp.where(iot == 1, max_v,
          jnp.where(iot == 2, _bcast(v0, 0), _bcast(v0, 1))))
    pub_v[...] = row
    pltpu.sync_copy(pub_v, sh_f.at[pl.ds(wid * 16, 16)])
    plsc.subcore_barrier()
    pltpu.sync_copy(sh_f, lpart_v)
    gmin = _bfly(plsc.load_gather(lpart_v, [iot * 16]), jnp.minimum)
    gmax = _bfly(plsc.load_gather(lpart_v, [iot * 16 + 1]), jnp.maximum)
    r0 = lpart_v[pl.ds(0, 16)]
    b0 = _bcast(r0, 2)
    d0 = _bcast(r0, 3)

    mmne = gmin != gmax

    def l1(k, cnt):
        base = k * 16
        fb = 2 * (base + iot)
        b = plsc.load_gather(vals_v, [fb])
        d = plsc.load_gather(vals_v, [fb + 1])
        pid = wid * CPW + base + iot
        m = b != d
        m = jnp.where(pid == P, mmne, m)
        m = jnp.logical_and(m, pid <= P)
        mi = m.astype(I32)
        inc = plsc.cumsum(mi)
        ebuf[pl.ds(base, 16)] = cnt + inc - mi
        mbuf[pl.ds(base, 16)] = mi
        return cnt + _bcast(inc, 15)
    cnt = lax.fori_loop(0, CPW // 16, l1, jnp.zeros((16,), I32))

    pubc_v[...] = jnp.where(iot == 0, cnt, 0)
    pltpu.sync_copy(pubc_v, sh_c.at[pl.ds(wid * 16, 16)])
    pat = jnp.where(iot % 2 == 0, b0, d0)

    def fl(j, c):
        fillb[pl.ds(j * 16, 16)] = pat
        return c
    lax.fori_loop(0, VPW // 16, fl, 0)
    pltpu.sync_copy(fillb, ofin.at[pl.ds(wid * VPW, VPW)])
    plsc.subcore_barrier()

    pltpu.sync_copy(sh_c, lcnt_v)
    cvec = plsc.load_gather(lcnt_v, [iot * 16])
    goff = _bcast(plsc.cumsum(cvec) - cvec, wid)

    @pl.when(wid == OWNER)
    def _():
        pv = jnp.where(iot == 0, gmin, gmax)
        plsc.store_scatter(vals_v, [2 * LSLOT + iot], pv, mask=iot < 2)

    dmp = 2 * (P + 100 + wid)

    def l2(k, c):
        base = k * 16
        mi = mbuf[pl.ds(base, 16)]
        ex = ebuf[pl.ds(base, 16)]
        dest = 2 * (goff + ex)
        live = mi > 0
        fb = 2 * (base + iot)
        plsc.store_scatter(sidx_v, [fb], jnp.where(live, dest, dmp))
        plsc.store_scatter(sidx_v, [fb + 1],
                           jnp.where(live, dest + 1, dmp + 1))
        return c
    lax.fori_loop(0, CPW // 16, l2, 0)

    def fire_s(j, c):
        pltpu.async_copy(vals_v.at[pl.ds(j * 128, 128)],
                         ofin.at[sidx_v.at[pl.ds(j * 128, 128)]], ssem)
        return c
    lax.fori_loop(0, NCH, fire_s, 0)
    pltpu.make_async_copy(vals_v, ofin.at[pl.ds(0, VPW)], ssem).wait()

    @pl.when(wid == 0)
    def _():
        pltpu.make_async_copy(tab.at[pl.ds(0, 16)], ev_o, esem).wait()
        pltpu.sync_copy(ev_o, oess)


_kern = pl.kernel(
    _body,
    out_type=(jax.ShapeDtypeStruct((IPAD,), F32),
              jax.ShapeDtypeStruct((16,), F32)),
    mesh=plsc.VectorSubcoreMesh(core_axis_name="c", subcore_axis_name="s",
                                num_cores=1),
    compiler_params=pltpu.CompilerParams(needs_layout_passes=False),
    scratch_types=[
        pltpu.VMEM((VPW,), I32),
        pltpu.VMEM((VPW,), F32),
        pltpu.VMEM((VPW,), I32),
        pltpu.VMEM((RED,), F32),
        pltpu.VMEM((CPW,), I32),
        pltpu.VMEM((CPW,), I32),
        pltpu.VMEM((VPW,), F32),
        pltpu.VMEM((16,), I32),
        pltpu.VMEM((16,), F32),
        pltpu.VMEM((16,), F32),
        pltpu.VMEM((256,), F32),
        pltpu.VMEM((16,), I32),
        pltpu.VMEM((256,), I32),
        pltpu.VMEM_SHARED((256,), F32),
        pltpu.VMEM_SHARED((256,), I32),
        pltpu.SemaphoreType.DMA,
        pltpu.SemaphoreType.DMA,
        pltpu.SemaphoreType.DMA,
    ],
)


@jax.jit
def kernel(filtration_values, finite_indices, essential_indices):
    tab = jnp.concatenate(
        [filtration_values, jnp.zeros((NPAD - N,), F32)])
    fidx = jnp.concatenate(
        [finite_indices, jnp.zeros((IPAD - 2 * P,), I32)])
    eidx = jnp.concatenate(
        [essential_indices, jnp.zeros((12,), I32)])
    ofin, oess = _kern(tab, fidx, eidx)
    return ofin[:2 * (P + 1)].reshape(P + 1, 2), oess[:4].reshape(4, 1)
